# baseline (speedup 1.0000x reference)
import functools

import jax
import jax.numpy as jnp
from jax import lax
from jax.experimental import pallas as pl
from jax.experimental.pallas import tpu as pltpu
from jax.experimental.pallas import tpu_sc as plsc

N, D, H, E = 8192, 1024, 2048, 16
B = 128
NB = N // B + E
NPAD = NB * B

EP = 128
NW = 32
TPW = N // NW
CH = 32
NCH = TPW // CH
NBUF = 3
GCH = 128
GNCH = TPW // GCH


def _routing_kernel(t_ref, slot_ref, blk_exp_ref, act_ref):
    t = t_ref[...]
    lane = lax.broadcasted_iota(jnp.int32, (N, E), 1)
    m = jnp.max(t, axis=1, keepdims=True)
    cand = jnp.where(t == m, lane, E)
    ct = jnp.min(cand, axis=1, keepdims=True)
    onehot = (lane == ct).astype(jnp.float32)

    c = onehot
    k = 1
    while k < N:
        c = c + jnp.concatenate(
            [jnp.zeros((k, E), jnp.float32), c[: N - k, :]], axis=0)
        k *= 2
    rank = jnp.sum(onehot * c, axis=1, keepdims=True) - 1.0
    counts = c[N - 1 : N, :].astype(jnp.int32)
    nb = (counts + (B - 1)) // B

    nb_b = jnp.broadcast_to(nb, (E, E))
    r_i = lax.broadcasted_iota(jnp.int32, (E, E), 0)
    c_j = lax.broadcasted_iota(jnp.int32, (E, E), 1)
    excl_col = jnp.sum(jnp.where(c_j < r_i, nb_b, 0), axis=1,
                       keepdims=True)
    excl_row = jnp.sum(jnp.where(r_i == c_j,
                                 jnp.broadcast_to(excl_col, (E, E)), 0),
                       axis=0, keepdims=True)

    sel_start = jnp.sum(onehot * excl_row.astype(jnp.float32), axis=1,
                        keepdims=True)
    slot_ref[...] = (sel_start * B + rank).astype(jnp.int32)

    total_used = jnp.sum(nb, axis=1, keepdims=True)
    bi = lax.broadcasted_iota(jnp.int32, (NB, E), 0)
    bi_eff = jnp.minimum(bi, total_used - 1)
    excl_b = jnp.broadcast_to(excl_row, (NB, E))
    blk_exp_ref[...] = jnp.sum(
        jnp.where(excl_b <= bi_eff, 1, 0), axis=1, keepdims=True) - 1
    bi1 = lax.broadcasted_iota(jnp.int32, (NB, 1), 0)
    act_ref[...] = (bi1 < total_used).astype(jnp.int32)


def _mlp_kernel(be_ref, act_ref, x_ref, w1_ref, b1_ref, w2_ref, b2_ref, o_ref):
    del be_ref

    @pl.when(act_ref[pl.program_id(0)] == 1)
    def _():
        h = jnp.maximum(
            jnp.dot(x_ref[...], w1_ref[0], preferred_element_type=jnp.float32)
            + b1_ref[0], 0.0)
        o_ref[:, :E] = (
            jnp.dot(h, w2_ref[0], preferred_element_type=jnp.float32)
            + b2_ref[0])


@functools.lru_cache(maxsize=1)
def _build_sc_kernels():
    mesh = plsc.VectorSubcoreMesh(
        core_axis_name="c", subcore_axis_name="s", num_cores=2, num_subcores=16)

    @functools.partial(
        pl.kernel,
        out_type=jax.ShapeDtypeStruct((NPAD, D), jnp.float32),
        mesh=mesh,
        compiler_params=pltpu.CompilerParams(use_tc_tiling_on_sc=True),
        scratch_types=[
            pltpu.VMEM((NCH, CH), jnp.int32),
            pltpu.VMEM((NBUF, CH, D), jnp.float32),
            pltpu.SemaphoreType.DMA,
            pltpu.SemaphoreType.DMA,
            pltpu.SemaphoreType.DMA,
            pltpu.SemaphoreType.DMA,
            pltpu.SemaphoreType.DMA,
            pltpu.SemaphoreType.DMA,
        ],
    )
    def scatter_x(x_hbm, slot_hbm, xs_hbm, idx_v, bufs, l0, l1, l2, s0, s1, s2):
        lsem = [l0, l1, l2]
        ssem = [s0, s1, s2]
        wid = lax.axis_index("s") * 2 + lax.axis_index("c")
        base = wid * TPW
        pltpu.sync_copy(slot_hbm.at[pl.ds(wid * NCH, NCH)], idx_v)
        loads = [None] * NCH
        scats = [None] * NCH

        def start_load(j):
            b = j % NBUF
            loads[j] = pltpu.async_copy(
                x_hbm.at[pl.ds(base + j * CH, CH)], bufs.at[b], lsem[b])

        start_load(0)
        start_load(1)
        for j in range(NCH):
            b = j % NBUF
            loads[j].wait()
            scats[j] = pltpu.async_copy(bufs.at[b], xs_hbm.at[idx_v.at[j]],
                                        ssem[b])
            if j + 2 < NCH:
                if j >= 1:
                    scats[j - 1].wait()
                start_load(j + 2)
        for j in range(NCH - 3, NCH):
            scats[j].wait()

    @functools.partial(
        pl.kernel,
        out_type=jax.ShapeDtypeStruct((N, EP), jnp.float32),
        mesh=mesh,
        compiler_params=pltpu.CompilerParams(use_tc_tiling_on_sc=True),
        scratch_types=[
            pltpu.VMEM((GNCH, GCH), jnp.int32),
            pltpu.VMEM((GCH, EP), jnp.float32),
            pltpu.SemaphoreType.DMA,
        ],
    )
    def gather_out(ys_hbm, slot_hbm, out_hbm, idx_v, rows_v, sem):
        wid = lax.axis_index("s") * 2 + lax.axis_index("c")
        base = wid * TPW
        pltpu.sync_copy(slot_hbm.at[pl.ds(wid * GNCH, GNCH)], idx_v)
        for j in range(GNCH):
            pltpu.async_copy(ys_hbm.at[idx_v.at[j]], rows_v, sem).wait()
            pltpu.sync_copy(rows_v, out_hbm.at[pl.ds(base + j * GCH, GCH)])

    return scatter_x, gather_out


def kernel(x, ten_CT, W1, b1, W2, b2):
    slot, blk_exp, act = pl.pallas_call(
        _routing_kernel,
        out_shape=(
            jax.ShapeDtypeStruct((N, 1), jnp.int32),
            jax.ShapeDtypeStruct((NB, 1), jnp.int32),
            jax.ShapeDtypeStruct((NB, 1), jnp.int32),
        ),
    )(ten_CT)
    slot_flat = slot.reshape(N)
    scatter_x, gather_out = _build_sc_kernels()

    x_sorted = scatter_x(x, slot_flat.reshape(N // CH, CH))

    grid_spec = pltpu.PrefetchScalarGridSpec(
        num_scalar_prefetch=2,
        grid=(NB,),
        in_specs=[
            pl.BlockSpec((B, D), lambda i, be, act: (i * act[i], 0)),
            pl.BlockSpec((1, D, H), lambda i, be, act: (be[i], 0, 0)),
            pl.BlockSpec((1, 1, H), lambda i, be, act: (be[i], 0, 0)),
            pl.BlockSpec((1, H, E), lambda i, be, act: (be[i], 0, 0)),
            pl.BlockSpec((1, 1, E), lambda i, be, act: (be[i], 0, 0)),
        ],
        out_specs=pl.BlockSpec((B, EP), lambda i, be, act: (i, 0)),
    )
    y_sorted = pl.pallas_call(
        _mlp_kernel,
        grid_spec=grid_spec,
        out_shape=jax.ShapeDtypeStruct((NPAD, EP), jnp.float32),
    )(blk_exp.reshape(NB), act.reshape(NB), x_sorted, W1,
      b1.reshape(E, 1, H), W2, b2.reshape(E, 1, E))

    out_p = gather_out(y_sorted, slot_flat.reshape(N // GCH, GCH))
    return out_p[:, :E]


# speedup vs baseline: 1.1829x; 1.1829x over previous
import functools

import jax
import jax.numpy as jnp
from jax import lax
from jax.experimental import pallas as pl
from jax.experimental.pallas import tpu as pltpu
from jax.experimental.pallas import tpu_sc as plsc

N, D, H, E = 8192, 1024, 2048, 16
B = 512
NB = N // B + E
NPAD = NB * B

EP = 128
NW = 32
TPW = N // NW
CH = 32
NCH = TPW // CH
NBUF = 3
GCH = 128
GNCH = TPW // GCH


def _routing_kernel(t_ref, slot_ref, blk_exp_ref, act_ref):
    t = t_ref[...]
    lane = lax.broadcasted_iota(jnp.int32, (N, E), 1)
    m = jnp.max(t, axis=1, keepdims=True)
    cand = jnp.where(t == m, lane, E)
    ct = jnp.min(cand, axis=1, keepdims=True)
    onehot = (lane == ct).astype(jnp.float32)

    c = onehot
    k = 1
    while k < N:
        c = c + jnp.concatenate(
            [jnp.zeros((k, E), jnp.float32), c[: N - k, :]], axis=0)
        k *= 2
    rank = jnp.sum(onehot * c, axis=1, keepdims=True) - 1.0
    counts = c[N - 1 : N, :].astype(jnp.int32)
    nb = (counts + (B - 1)) // B

    nb_b = jnp.broadcast_to(nb, (E, E))
    r_i = lax.broadcasted_iota(jnp.int32, (E, E), 0)
    c_j = lax.broadcasted_iota(jnp.int32, (E, E), 1)
    excl_col = jnp.sum(jnp.where(c_j < r_i, nb_b, 0), axis=1,
                       keepdims=True)
    excl_row = jnp.sum(jnp.where(r_i == c_j,
                                 jnp.broadcast_to(excl_col, (E, E)), 0),
                       axis=0, keepdims=True)

    sel_start = jnp.sum(onehot * excl_row.astype(jnp.float32), axis=1,
                        keepdims=True)
    slot_ref[...] = (sel_start * B + rank).astype(jnp.int32)

    total_used = jnp.sum(nb, axis=1, keepdims=True)
    bi = lax.broadcasted_iota(jnp.int32, (NB, E), 0)
    bi_eff = jnp.minimum(bi, total_used - 1)
    excl_b = jnp.broadcast_to(excl_row, (NB, E))
    blk_exp_ref[...] = jnp.sum(
        jnp.where(excl_b <= bi_eff, 1, 0), axis=1, keepdims=True) - 1
    bi1 = lax.broadcasted_iota(jnp.int32, (NB, 1), 0)
    act_ref[...] = (bi1 < total_used).astype(jnp.int32)


def _mlp_kernel(be_ref, act_ref, x_ref, w1_ref, b1_ref, w2_ref, b2_ref, o_ref):
    del be_ref

    @pl.when(act_ref[pl.program_id(0)] == 1)
    def _():
        h = jnp.maximum(
            jnp.dot(x_ref[...], w1_ref[0], preferred_element_type=jnp.float32)
            + b1_ref[0], 0.0)
        o_ref[:, :E] = (
            jnp.dot(h, w2_ref[0], preferred_element_type=jnp.float32)
            + b2_ref[0])


@functools.lru_cache(maxsize=1)
def _build_sc_kernels():
    mesh = plsc.VectorSubcoreMesh(
        core_axis_name="c", subcore_axis_name="s", num_cores=2, num_subcores=16)

    @functools.partial(
        pl.kernel,
        out_type=jax.ShapeDtypeStruct((NPAD, D), jnp.float32),
        mesh=mesh,
        compiler_params=pltpu.CompilerParams(use_tc_tiling_on_sc=True),
        scratch_types=[
            pltpu.VMEM((NCH, CH), jnp.int32),
            pltpu.VMEM((NBUF, CH, D), jnp.float32),
            pltpu.SemaphoreType.DMA,
            pltpu.SemaphoreType.DMA,
            pltpu.SemaphoreType.DMA,
            pltpu.SemaphoreType.DMA,
            pltpu.SemaphoreType.DMA,
            pltpu.SemaphoreType.DMA,
        ],
    )
    def scatter_x(x_hbm, slot_hbm, xs_hbm, idx_v, bufs, l0, l1, l2, s0, s1, s2):
        lsem = [l0, l1, l2]
        ssem = [s0, s1, s2]
        wid = lax.axis_index("s") * 2 + lax.axis_index("c")
        base = wid * TPW
        pltpu.sync_copy(slot_hbm.at[pl.ds(wid * NCH, NCH)], idx_v)
        loads = [None] * NCH
        scats = [None] * NCH

        def start_load(j):
            b = j % NBUF
            loads[j] = pltpu.async_copy(
                x_hbm.at[pl.ds(base + j * CH, CH)], bufs.at[b], lsem[b])

        start_load(0)
        start_load(1)
        for j in range(NCH):
            b = j % NBUF
            loads[j].wait()
            scats[j] = pltpu.async_copy(bufs.at[b], xs_hbm.at[idx_v.at[j]],
                                        ssem[b])
            if j + 2 < NCH:
                if j >= 1:
                    scats[j - 1].wait()
                start_load(j + 2)
        for j in range(NCH - 3, NCH):
            scats[j].wait()

    @functools.partial(
        pl.kernel,
        out_type=jax.ShapeDtypeStruct((N, EP), jnp.float32),
        mesh=mesh,
        compiler_params=pltpu.CompilerParams(use_tc_tiling_on_sc=True),
        scratch_types=[
            pltpu.VMEM((GNCH, GCH), jnp.int32),
            pltpu.VMEM((GCH, EP), jnp.float32),
            pltpu.SemaphoreType.DMA,
        ],
    )
    def gather_out(ys_hbm, slot_hbm, out_hbm, idx_v, rows_v, sem):
        wid = lax.axis_index("s") * 2 + lax.axis_index("c")
        base = wid * TPW
        pltpu.sync_copy(slot_hbm.at[pl.ds(wid * GNCH, GNCH)], idx_v)
        for j in range(GNCH):
            pltpu.async_copy(ys_hbm.at[idx_v.at[j]], rows_v, sem).wait()
            pltpu.sync_copy(rows_v, out_hbm.at[pl.ds(base + j * GCH, GCH)])

    return scatter_x, gather_out


def kernel(x, ten_CT, W1, b1, W2, b2):
    slot, blk_exp, act = pl.pallas_call(
        _routing_kernel,
        out_shape=(
            jax.ShapeDtypeStruct((N, 1), jnp.int32),
            jax.ShapeDtypeStruct((NB, 1), jnp.int32),
            jax.ShapeDtypeStruct((NB, 1), jnp.int32),
        ),
    )(ten_CT)
    slot_flat = slot.reshape(N)
    scatter_x, gather_out = _build_sc_kernels()

    x_sorted = scatter_x(x, slot_flat.reshape(N // CH, CH))

    grid_spec = pltpu.PrefetchScalarGridSpec(
        num_scalar_prefetch=2,
        grid=(NB,),
        in_specs=[
            pl.BlockSpec((B, D), lambda i, be, act: (i * act[i], 0)),
            pl.BlockSpec((1, D, H), lambda i, be, act: (be[i], 0, 0)),
            pl.BlockSpec((1, 1, H), lambda i, be, act: (be[i], 0, 0)),
            pl.BlockSpec((1, H, E), lambda i, be, act: (be[i], 0, 0)),
            pl.BlockSpec((1, 1, E), lambda i, be, act: (be[i], 0, 0)),
        ],
        out_specs=pl.BlockSpec((B, EP), lambda i, be, act: (i, 0)),
    )
    y_sorted = pl.pallas_call(
        _mlp_kernel,
        grid_spec=grid_spec,
        out_shape=jax.ShapeDtypeStruct((NPAD, EP), jnp.float32),
    )(blk_exp.reshape(NB), act.reshape(NB), x_sorted, W1,
      b1.reshape(E, 1, H), W2, b2.reshape(E, 1, E))

    out_p = gather_out(y_sorted, slot_flat.reshape(N // GCH, GCH))
    return out_p[:, :E]


# speedup vs baseline: 1.1895x; 1.0056x over previous
import functools

import jax
import jax.numpy as jnp
from jax import lax
from jax.experimental import pallas as pl
from jax.experimental.pallas import tpu as pltpu
from jax.experimental.pallas import tpu_sc as plsc

N, D, H, E = 8192, 1024, 2048, 16
B = 512
NB = N // B + E
NPAD = NB * B

EP = 128
NW = 32
TPW = N // NW
CH = 16
NCH = TPW // CH
NBUF = 6
GCH = 128
GNCH = TPW // GCH


def _routing_kernel(t_ref, slot_ref, blk_exp_ref, act_ref):
    t = t_ref[...]
    lane = lax.broadcasted_iota(jnp.int32, (N, E), 1)
    m = jnp.max(t, axis=1, keepdims=True)
    cand = jnp.where(t == m, lane, E)
    ct = jnp.min(cand, axis=1, keepdims=True)
    onehot = (lane == ct).astype(jnp.float32)

    c = onehot
    k = 1
    while k < N:
        c = c + jnp.concatenate(
            [jnp.zeros((k, E), jnp.float32), c[: N - k, :]], axis=0)
        k *= 2
    rank = jnp.sum(onehot * c, axis=1, keepdims=True) - 1.0
    counts = c[N - 1 : N, :].astype(jnp.int32)
    nb = (counts + (B - 1)) // B

    nb_b = jnp.broadcast_to(nb, (E, E))
    r_i = lax.broadcasted_iota(jnp.int32, (E, E), 0)
    c_j = lax.broadcasted_iota(jnp.int32, (E, E), 1)
    excl_col = jnp.sum(jnp.where(c_j < r_i, nb_b, 0), axis=1,
                       keepdims=True)
    excl_row = jnp.sum(jnp.where(r_i == c_j,
                                 jnp.broadcast_to(excl_col, (E, E)), 0),
                       axis=0, keepdims=True)

    sel_start = jnp.sum(onehot * excl_row.astype(jnp.float32), axis=1,
                        keepdims=True)
    slot_ref[...] = (sel_start * B + rank).astype(jnp.int32)

    total_used = jnp.sum(nb, axis=1, keepdims=True)
    bi = lax.broadcasted_iota(jnp.int32, (NB, E), 0)
    bi_eff = jnp.minimum(bi, total_used - 1)
    excl_b = jnp.broadcast_to(excl_row, (NB, E))
    blk_exp_ref[...] = jnp.sum(
        jnp.where(excl_b <= bi_eff, 1, 0), axis=1, keepdims=True) - 1
    bi1 = lax.broadcasted_iota(jnp.int32, (NB, 1), 0)
    act_ref[...] = (bi1 < total_used).astype(jnp.int32)


def _mlp_kernel(be_ref, act_ref, x_ref, w1_ref, b1_ref, w2_ref, b2_ref, o_ref):
    del be_ref

    @pl.when(act_ref[pl.program_id(0)] == 1)
    def _():
        h = jnp.maximum(
            jnp.dot(x_ref[...], w1_ref[0], preferred_element_type=jnp.float32)
            + b1_ref[0], 0.0)
        o_ref[:, :E] = (
            jnp.dot(h, w2_ref[0], preferred_element_type=jnp.float32)
            + b2_ref[0])


@functools.lru_cache(maxsize=1)
def _build_sc_kernels():
    mesh = plsc.VectorSubcoreMesh(
        core_axis_name="c", subcore_axis_name="s", num_cores=2, num_subcores=16)

    @functools.partial(
        pl.kernel,
        out_type=jax.ShapeDtypeStruct((NPAD, D), jnp.float32),
        mesh=mesh,
        compiler_params=pltpu.CompilerParams(use_tc_tiling_on_sc=True),
        scratch_types=[
            pltpu.VMEM((NCH, CH), jnp.int32),
            pltpu.VMEM((NBUF, CH, D), jnp.float32),
        ] + [pltpu.SemaphoreType.DMA] * (2 * NBUF),
    )
    def scatter_x(x_hbm, slot_hbm, xs_hbm, idx_v, bufs, *sems):
        lsem = sems[:NBUF]
        ssem = sems[NBUF:]
        wid = lax.axis_index("s") * 2 + lax.axis_index("c")
        base = wid * TPW
        pltpu.sync_copy(slot_hbm.at[pl.ds(wid * NCH, NCH)], idx_v)
        loads = [None] * NCH
        scats = [None] * NCH

        def start_load(j):
            b = j % NBUF
            loads[j] = pltpu.async_copy(
                x_hbm.at[pl.ds(base + j * CH, CH)], bufs.at[b], lsem[b])

        for j in range(NBUF - 1):
            start_load(j)
        for j in range(NCH):
            b = j % NBUF
            loads[j].wait()
            scats[j] = pltpu.async_copy(bufs.at[b], xs_hbm.at[idx_v.at[j]],
                                        ssem[b])
            if j + NBUF - 1 < NCH:
                if j >= 1:
                    scats[j - 1].wait()
                start_load(j + NBUF - 1)
        for j in range(NCH - NBUF, NCH):
            scats[j].wait()

    @functools.partial(
        pl.kernel,
        out_type=jax.ShapeDtypeStruct((N, EP), jnp.float32),
        mesh=mesh,
        compiler_params=pltpu.CompilerParams(use_tc_tiling_on_sc=True),
        scratch_types=[
            pltpu.VMEM((GNCH, GCH), jnp.int32),
            pltpu.VMEM((GNCH, GCH, EP), jnp.float32),
            pltpu.SemaphoreType.DMA,
            pltpu.SemaphoreType.DMA,
        ],
    )
    def gather_out(ys_hbm, slot_hbm, out_hbm, idx_v, rows_v, g0, g1):
        gsem = [g0, g1]
        wid = lax.axis_index("s") * 2 + lax.axis_index("c")
        base = wid * TPW
        pltpu.sync_copy(slot_hbm.at[pl.ds(wid * GNCH, GNCH)], idx_v)
        gets = [pltpu.async_copy(ys_hbm.at[idx_v.at[j]], rows_v.at[j], gsem[j])
                for j in range(GNCH)]
        for j in range(GNCH):
            gets[j].wait()
            pltpu.sync_copy(rows_v.at[j],
                            out_hbm.at[pl.ds(base + j * GCH, GCH)])

    return scatter_x, gather_out


def kernel(x, ten_CT, W1, b1, W2, b2):
    slot, blk_exp, act = pl.pallas_call(
        _routing_kernel,
        out_shape=(
            jax.ShapeDtypeStruct((N, 1), jnp.int32),
            jax.ShapeDtypeStruct((NB, 1), jnp.int32),
            jax.ShapeDtypeStruct((NB, 1), jnp.int32),
        ),
    )(ten_CT)
    slot_flat = slot.reshape(N)
    scatter_x, gather_out = _build_sc_kernels()

    x_sorted = scatter_x(x, slot_flat.reshape(N // CH, CH))

    grid_spec = pltpu.PrefetchScalarGridSpec(
        num_scalar_prefetch=2,
        grid=(NB,),
        in_specs=[
            pl.BlockSpec((B, D), lambda i, be, act: (i * act[i], 0)),
            pl.BlockSpec((1, D, H), lambda i, be, act: (be[i], 0, 0)),
            pl.BlockSpec((1, 1, H), lambda i, be, act: (be[i], 0, 0)),
            pl.BlockSpec((1, H, E), lambda i, be, act: (be[i], 0, 0)),
            pl.BlockSpec((1, 1, E), lambda i, be, act: (be[i], 0, 0)),
        ],
        out_specs=pl.BlockSpec((B, EP), lambda i, be, act: (i, 0)),
    )
    y_sorted = pl.pallas_call(
        _mlp_kernel,
        grid_spec=grid_spec,
        out_shape=jax.ShapeDtypeStruct((NPAD, EP), jnp.float32),
    )(blk_exp.reshape(NB), act.reshape(NB), x_sorted, W1,
      b1.reshape(E, 1, H), W2, b2.reshape(E, 1, E))

    out_p = gather_out(y_sorted, slot_flat.reshape(N // GCH, GCH))
    return out_p[:, :E]
